# baseline (speedup 1.0000x reference)
import functools

import jax
import jax.numpy as jnp
from jax import lax
from jax.experimental import pallas as pl
from jax.experimental.pallas import tpu as pltpu
from jax.experimental.pallas import tpu_sc as plsc

VOCAB = 1000000
HDIM = 16
BATCH = 16384

_PACK = 128 // HDIM
_NUM_CORES = 2
_NUM_SUBCORES = 16
_NW = _NUM_CORES * _NUM_SUBCORES
_B_PER_W = BATCH // _NW
_CHUNK = 128
_N_CHUNKS = _B_PER_W // _CHUNK
_GROUPS_PER_CHUNK = _CHUNK // 16

_mesh = plsc.VectorSubcoreMesh(core_axis_name="c", subcore_axis_name="s")


@functools.partial(
    pl.kernel,
    mesh=_mesh,
    out_type=jax.ShapeDtypeStruct((BATCH * HDIM,), jnp.float32),
    scratch_types=[
        pltpu.VMEM((_B_PER_W,), jnp.int32),
        pltpu.VMEM((_N_CHUNKS, _CHUNK), jnp.int32),
        pltpu.VMEM((_B_PER_W, 128), jnp.float32),
        pltpu.VMEM((_B_PER_W * HDIM,), jnp.float32),
        pltpu.SemaphoreType.DMA,
    ],
    compiler_params=pltpu.CompilerParams(
        use_tc_tiling_on_sc=True, needs_layout_passes=False
    ),
)
def _gather_kernel(table_hbm, idx_hbm, out_hbm, idx_v, line_v, packed_v,
                   out_v, sem):
    wid = lax.axis_index("s") * _NUM_CORES + lax.axis_index("c")
    base = wid * _B_PER_W
    pltpu.sync_copy(idx_hbm.at[pl.ds(base, _B_PER_W)], idx_v)

    lane = lax.iota(jnp.int32, 16)
    copies = []
    for c in range(_N_CHUNKS):
        for g8 in range(_GROUPS_PER_CHUNK):
            g = c * _GROUPS_PER_CHUNK + g8
            v = idx_v[pl.ds(g * 16, 16)]
            line_v.at[c][pl.ds(g8 * 16, 16)] = v >> 3
        copies.append(
            pltpu.async_copy(
                table_hbm.at[line_v.at[c]],
                packed_v.at[pl.ds(c * _CHUNK, _CHUNK)],
                sem,
            )
        )
    for c in range(_N_CHUNKS):
        copies[c].wait()
        for g8 in range(_GROUPS_PER_CHUNK):
            g = c * _GROUPS_PER_CHUNK + g8
            v = idx_v[pl.ds(g * 16, 16)]
            col0 = (v & (_PACK - 1)) << 4
            row = lane + g * 16
            o0 = row << 4
            for col in range(HDIM):
                vals = plsc.load_gather(packed_v, [row, col0 + col])
                plsc.store_scatter(out_v, [o0 + col], vals)

    pltpu.sync_copy(out_v, out_hbm.at[pl.ds(base * HDIM, _B_PER_W * HDIM)])


def kernel(ordered_embs, nodes):
    table2d = ordered_embs.reshape(VOCAB // _PACK, 128)
    idx1d = nodes.reshape(BATCH)
    out = _gather_kernel(table2d, idx1d)
    return out.reshape(BATCH, HDIM)


# speedup vs baseline: 6.0104x; 6.0104x over previous
import functools

import jax
import jax.numpy as jnp
from jax import lax
from jax.experimental import pallas as pl
from jax.experimental.pallas import tpu as pltpu
from jax.experimental.pallas import tpu_sc as plsc

VOCAB = 1000000
HDIM = 16
BATCH = 16384

_NUM_CORES = 2
_NUM_SUBCORES = 16
_NW = _NUM_CORES * _NUM_SUBCORES
_B_PER_W = BATCH // _NW
_G = 16
_NG = _B_PER_W // _G

_mesh = plsc.VectorSubcoreMesh(core_axis_name="c", subcore_axis_name="s")


@functools.partial(
    pl.kernel,
    mesh=_mesh,
    out_type=jax.ShapeDtypeStruct((HDIM, BATCH), jnp.float32),
    scratch_types=[
        pltpu.VMEM((_B_PER_W,), jnp.int32),
        pltpu.VMEM((2, _G, HDIM, 128), jnp.float32),
        pltpu.VMEM((HDIM, _B_PER_W), jnp.float32),
        pltpu.SemaphoreType.DMA,
        pltpu.SemaphoreType.DMA,
    ],
    compiler_params=pltpu.CompilerParams(
        use_tc_tiling_on_sc=True, needs_layout_passes=False
    ),
)
def _gather_kernel(table_hbm, idx_hbm, out_hbm, idx_v, blk_v, out_v,
                   sem0, sem1):
    wid = lax.axis_index("s") * _NUM_CORES + lax.axis_index("c")
    base = wid * _B_PER_W
    pltpu.sync_copy(idx_hbm.at[pl.ds(base, _B_PER_W)], idx_v)
    iota = lax.iota(jnp.int32, 16)
    zeros = jnp.full((16,), 0, jnp.int32)
    sems = (sem0, sem1)

    def fire(g, slot):
        v = idx_v[pl.ds(g * _G, _G)]
        c0v = (v >> 7) << 7
        for b in range(_G):
            col0 = jnp.sum(jnp.where(iota == b, c0v, 0))
            col0 = pl.multiple_of(col0, 128)
            pltpu.async_copy(
                table_hbm.at[:, pl.ds(col0, 128)],
                blk_v.at[slot, b],
                sems[slot],
            )

    def drain_and_extract(g, slot):
        for b in range(_G):
            pltpu.make_async_copy(
                table_hbm.at[:, pl.ds(0, 128)], blk_v.at[slot, b], sems[slot]
            ).wait()
        v = idx_v[pl.ds(g * _G, _G)]
        lanev = v & 127
        for b in range(_G):
            lane = jnp.sum(jnp.where(iota == b, lanev, 0))
            vals = plsc.load_gather(blk_v.at[slot, b], [iota, zeros + lane])
            plsc.store_scatter(out_v, [iota, zeros + (g * _G + b)], vals)

    fire(0, 0)
    fire(1, 1)

    def body(gp, carry):
        g0 = 2 * gp
        drain_and_extract(g0, 0)

        @pl.when(gp < _NG // 2 - 1)
        def _():
            fire(g0 + 2, 0)

        drain_and_extract(g0 + 1, 1)

        @pl.when(gp < _NG // 2 - 1)
        def _():
            fire(g0 + 3, 1)

        return carry

    lax.fori_loop(0, _NG // 2, body, 0)
    pltpu.sync_copy(out_v, out_hbm.at[:, pl.ds(base, _B_PER_W)])


def kernel(ordered_embs, nodes):
    table_t = ordered_embs.T
    idx1d = nodes.reshape(BATCH)
    out_t = _gather_kernel(table_t, idx1d)
    return out_t.T


# speedup vs baseline: 6.0249x; 1.0024x over previous
import functools

import jax
import jax.numpy as jnp
from jax import lax
from jax.experimental import pallas as pl
from jax.experimental.pallas import tpu as pltpu
from jax.experimental.pallas import tpu_sc as plsc

VOCAB = 1000000
HDIM = 16
BATCH = 16384

_NUM_CORES = 2
_NUM_SUBCORES = 16
_NW = _NUM_CORES * _NUM_SUBCORES
_B_PER_W = BATCH // _NW
_G = 16
_NG = _B_PER_W // _G

_mesh = plsc.VectorSubcoreMesh(core_axis_name="c", subcore_axis_name="s")


@functools.partial(
    pl.kernel,
    mesh=_mesh,
    out_type=jax.ShapeDtypeStruct((HDIM, BATCH), jnp.float32),
    scratch_types=[
        pltpu.VMEM((_B_PER_W,), jnp.int32),
        pltpu.VMEM((2, _G, 2, 8, 128), jnp.float32),
        pltpu.VMEM((HDIM, _B_PER_W), jnp.float32),
        pltpu.SemaphoreType.DMA,
        pltpu.SemaphoreType.DMA,
    ],
    compiler_params=pltpu.CompilerParams(
        use_tc_tiling_on_sc=True, needs_layout_passes=False
    ),
)
def _gather_kernel(table_hbm, idx_hbm, out_hbm, idx_v, blk_v, out_v,
                   sem0, sem1):
    wid = lax.axis_index("s") * _NUM_CORES + lax.axis_index("c")
    base = wid * _B_PER_W
    pltpu.sync_copy(idx_hbm.at[pl.ds(base, _B_PER_W)], idx_v)
    iota = lax.iota(jnp.int32, 16)
    zeros = jnp.full((16,), 0, jnp.int32)
    sems = (sem0, sem1)

    def fire(g, slot):
        v = idx_v[pl.ds(g * _G, _G)]
        c0v = (v >> 7) << 7
        for b in range(_G):
            col0 = jnp.sum(jnp.where(iota == b, c0v, 0))
            col0 = pl.multiple_of(col0, 128)
            for half in range(2):
                pltpu.async_copy(
                    table_hbm.at[pl.ds(half * 8, 8), pl.ds(col0, 128)],
                    blk_v.at[slot, b, half],
                    sems[slot],
                )

    def drain_and_extract(g, slot):
        for b in range(_G):
            for half in range(2):
                pltpu.make_async_copy(
                    table_hbm.at[pl.ds(half * 8, 8), pl.ds(0, 128)],
                    blk_v.at[slot, b, half],
                    sems[slot],
                ).wait()
        v = idx_v[pl.ds(g * _G, _G)]
        lanev = v & 127
        for b in range(_G):
            lane = jnp.sum(jnp.where(iota == b, lanev, 0))
            vals = plsc.load_gather(
                blk_v.at[slot, b], [iota >> 3, iota & 7, zeros + lane]
            )
            plsc.store_scatter(out_v, [iota, zeros + (g * _G + b)], vals)

    fire(0, 0)
    fire(1, 1)

    def body(gp, carry):
        g0 = 2 * gp
        drain_and_extract(g0, 0)

        @pl.when(gp < _NG // 2 - 1)
        def _():
            fire(g0 + 2, 0)

        drain_and_extract(g0 + 1, 1)

        @pl.when(gp < _NG // 2 - 1)
        def _():
            fire(g0 + 3, 1)

        return carry

    lax.fori_loop(0, _NG // 2, body, 0)
    pltpu.sync_copy(out_v, out_hbm.at[:, pl.ds(base, _B_PER_W)])


def kernel(ordered_embs, nodes):
    table_t = ordered_embs.T
    idx1d = nodes.reshape(BATCH)
    out_t = _gather_kernel(table_t, idx1d)
    return out_t.T
